# baseline (speedup 1.0000x reference)
import functools
import math

import jax
import jax.numpy as jnp
from jax import lax
from jax.experimental import pallas as pl
from jax.experimental.pallas import tpu as pltpu
from jax.experimental.pallas import tpu_sc as plsc

EMBED_DIM = 768
NUM_HEADS = 12
HEAD_DIM = 64
NUM_NODES = 8
ROPE_BASE = 10000.0

QBLK = 256

_TWO_PI_HI = 6.28125
_TWO_PI_LO = 0.0019353071795864769
_INV_TWO_PI = 1.0 / (2.0 * math.pi)

_COS_COEF = [1.0 / math.factorial(2 * m) * (-1) ** m for m in range(9)]
_SIN_COEF = [1.0 / math.factorial(2 * m + 1) * (-1) ** m for m in range(9)]


def _cos_sin(x):
    f32 = jnp.float32
    half = x.shape[1]
    x2 = jnp.concatenate([x, x], axis=1)
    u = x2 * _INV_TWO_PI
    kq = jnp.floor(u + 0.5)
    r = (x2 - kq * _TWO_PI_HI) - kq * _TWO_PI_LO
    y = r * r
    lane = jax.lax.broadcasted_iota(jnp.int32, (1, 2 * half), 1)
    is_cos = lane < half
    coef = [jnp.where(is_cos, _COS_COEF[m], _SIN_COEF[m]) for m in range(9)]
    p = jnp.broadcast_to(coef[8], y.shape)
    for m in range(7, -1, -1):
        p = p * y + coef[m]
    p = p * jnp.where(is_cos, 1.0, r)
    return p[:, :half].astype(f32), p[:, half:].astype(f32)


def _table_kernel(tab_ref):
    f32 = jnp.float32
    n = tab_ref.shape[0]
    hd = HEAD_DIM
    half = hd // 2
    t = jax.lax.broadcasted_iota(jnp.int32, (n, 1), 0).astype(f32)
    i2 = jax.lax.broadcasted_iota(jnp.int32, (1, half), 1).astype(f32)
    inv_freq = jnp.exp(i2 * (-2.0 * math.log(ROPE_BASE) / hd))
    cos, sin = _cos_sin(t * inv_freq)
    tab_ref[...] = jnp.concatenate([cos, cos, sin, sin], axis=1)


def _proj_route_kernel(x_ref, wqkv_ref, wr_ref,
                       qk_out, vx_out, nc_out, nr_out, pos_out):
    f32 = jnp.float32
    bf16 = jnp.bfloat16
    x = x_ref[...]
    xb = x.astype(bf16)
    n = x.shape[0]
    K = wr_ref.shape[1]
    hd = HEAD_DIM

    qkv = jax.lax.dot_general(xb, wqkv_ref[0].astype(bf16),
                              (((1,), (1,)), ((), ())),
                              preferred_element_type=f32)
    logits_t = jax.lax.dot_general(wr_ref[0], x, (((1,), (1,)), ((), ())),
                                   preferred_element_type=f32)

    kidx = jax.lax.broadcasted_iota(jnp.int32, (K, n), 0).astype(f32)
    mx = jnp.max(logits_t, axis=0, keepdims=True)
    node_t = jnp.min(jnp.where(logits_t == mx, kidx, float(K)),
                     axis=0, keepdims=True)
    onehot_t = (kidx == node_t).astype(f32)
    cum = onehot_t
    shift = 1
    while shift < n:
        zeros = jnp.zeros((K, shift), dtype=f32)
        cum = cum + jnp.concatenate([zeros, cum[:, :-shift]], axis=1)
        shift *= 2
    pos_t = jnp.sum(onehot_t * cum, axis=0, keepdims=True) - 1.0

    qk_out[0] = qkv[:, :2 * hd]
    vx_out[0] = jnp.concatenate([qkv[:, 2 * hd:], jnp.ones((n, hd), f32)],
                                axis=1).astype(bf16)
    nr_out[0] = node_t
    nc_out[0] = jnp.transpose(node_t)
    pos_out[0] = pos_t.astype(jnp.int32)


def _make_sc_gather(n_rows, width, n_workers):
    rows_per_w = n_rows // n_workers
    mesh = plsc.VectorSubcoreMesh(core_axis_name="c", subcore_axis_name="s")

    @functools.partial(
        pl.kernel, mesh=mesh,
        out_type=jax.ShapeDtypeStruct((n_rows, width), jnp.float32),
        scratch_types=[
            pltpu.VMEM((rows_per_w,), jnp.int32),
            pltpu.VMEM((rows_per_w, width), jnp.float32),
            pltpu.SemaphoreType.DMA,
        ],
    )
    def sc_gather(table_hbm, idx_hbm, out_hbm, idx_v, rows_v, sem):
        wid = lax.axis_index("s") * 2 + lax.axis_index("c")
        base = wid * rows_per_w
        pltpu.sync_copy(idx_hbm.at[pl.ds(base, rows_per_w)], idx_v)
        pltpu.async_copy(table_hbm.at[idx_v], rows_v, sem).wait()
        pltpu.sync_copy(rows_v, out_hbm.at[pl.ds(base, rows_per_w)])

    return sc_gather


def _rope_apply_kernel(qk_ref, cs_ref, q_out, k_out):
    bf16 = jnp.bfloat16
    hd = HEAD_DIM
    half = hd // 2
    scale = hd ** -0.5
    qk = qk_ref[0]
    cs = cs_ref[0]
    cos2 = cs[:, :hd]
    sin2 = cs[:, hd:]
    cos4 = jnp.concatenate([cos2, cos2], axis=1)
    sin4 = jnp.concatenate([sin2, sin2], axis=1)
    rot = jnp.concatenate([-qk[:, half:hd], qk[:, :half],
                           -qk[:, hd + half:], qk[:, hd:hd + half]], axis=1)
    qk_roped = qk * cos4 + rot * sin4
    q_out[0] = (qk_roped[:, :hd] * scale).astype(bf16)
    k_out[0] = qk_roped[:, hd:].astype(bf16)


def _attn_out_kernel(q_ref, k_ref, vx_ref, nc_ref, nr_ref, wo_ref,
                     o_ref, y_ref):
    f32 = jnp.float32
    bf16 = jnp.bfloat16
    qi = pl.program_id(0)
    H = q_ref.shape[0]
    n = k_ref.shape[1]
    hd = HEAD_DIM

    rr = jax.lax.broadcasted_iota(jnp.int32, (QBLK, n), 0) + qi * QBLK
    cc = jax.lax.broadcasted_iota(jnp.int32, (QBLK, n), 1)
    causal = rr >= cc

    for h in range(H):
        s = jax.lax.dot_general(q_ref[h], k_ref[h], (((1,), (1,)), ((), ())),
                                preferred_element_type=f32)
        d = nc_ref[h] - nr_ref[h]
        e = jnp.exp(s + d * d * (-1e9))
        e = jnp.where(causal, e, 0.0)
        accx = jax.lax.dot_general(e.astype(bf16), vx_ref[h],
                                   (((1,), (0,)), ((), ())),
                                   preferred_element_type=f32)
        y_ref[:, h * hd:(h + 1) * hd] = (
            accx[:, :hd] / accx[:, hd:hd + 1]).astype(bf16)

    o_ref[...] = jax.lax.dot_general(y_ref[...], wo_ref[...],
                                     (((1,), (1,)), ((), ())),
                                     preferred_element_type=f32)


@jax.jit
def kernel(x, Wq, Wk, Wv, Wr, Wo):
    B, N, D = x.shape
    H, hd, K = NUM_HEADS, HEAD_DIM, NUM_NODES
    x2 = x.reshape(N, D)
    wqkv = jnp.concatenate([Wq.reshape(H, hd, D), Wk.reshape(H, hd, D),
                            Wv.reshape(H, hd, D)], axis=1)

    table = pl.pallas_call(
        _table_kernel,
        out_shape=jax.ShapeDtypeStruct((N, 2 * hd), jnp.float32),
    )()

    qk2, vx, node_c, node_r, pos = pl.pallas_call(
        _proj_route_kernel,
        grid=(H,),
        in_specs=[
            pl.BlockSpec((N, D), lambda h: (0, 0)),
            pl.BlockSpec((1, 3 * hd, D), lambda h: (h, 0, 0)),
            pl.BlockSpec((1, K, D), lambda h: (h, 0, 0)),
        ],
        out_specs=[
            pl.BlockSpec((1, N, 2 * hd), lambda h: (h, 0, 0)),
            pl.BlockSpec((1, N, 2 * hd), lambda h: (h, 0, 0)),
            pl.BlockSpec((1, N, 1), lambda h: (h, 0, 0)),
            pl.BlockSpec((1, 1, N), lambda h: (h, 0, 0)),
            pl.BlockSpec((1, 1, N), lambda h: (h, 0, 0)),
        ],
        out_shape=[
            jax.ShapeDtypeStruct((H, N, 2 * hd), jnp.float32),
            jax.ShapeDtypeStruct((H, N, 2 * hd), jnp.bfloat16),
            jax.ShapeDtypeStruct((H, N, 1), jnp.float32),
            jax.ShapeDtypeStruct((H, 1, N), jnp.float32),
            jax.ShapeDtypeStruct((H, 1, N), jnp.int32),
        ],
    )(x2, wqkv, Wr.reshape(H, K, D))

    sc_gather = _make_sc_gather(H * N, 2 * hd, 32)
    cossin = sc_gather(table, pos.reshape(H * N))

    q, k = pl.pallas_call(
        _rope_apply_kernel,
        grid=(H,),
        in_specs=[
            pl.BlockSpec((1, N, 2 * hd), lambda h: (h, 0, 0)),
            pl.BlockSpec((1, N, 2 * hd), lambda h: (h, 0, 0)),
        ],
        out_specs=[
            pl.BlockSpec((1, N, hd), lambda h: (h, 0, 0)),
            pl.BlockSpec((1, N, hd), lambda h: (h, 0, 0)),
        ],
        out_shape=[
            jax.ShapeDtypeStruct((H, N, hd), jnp.bfloat16),
            jax.ShapeDtypeStruct((H, N, hd), jnp.bfloat16),
        ],
    )(qk2, cossin.reshape(H, N, 2 * hd))

    out = pl.pallas_call(
        _attn_out_kernel,
        grid=(N // QBLK,),
        in_specs=[
            pl.BlockSpec((H, QBLK, hd), lambda i: (0, i, 0)),
            pl.BlockSpec((H, N, hd), lambda i: (0, 0, 0)),
            pl.BlockSpec((H, N, 2 * hd), lambda i: (0, 0, 0)),
            pl.BlockSpec((H, QBLK, 1), lambda i: (0, i, 0)),
            pl.BlockSpec((H, 1, N), lambda i: (0, 0, 0)),
            pl.BlockSpec((D, H * hd), lambda i: (0, 0)),
        ],
        out_specs=pl.BlockSpec((QBLK, D), lambda i: (i, 0)),
        out_shape=jax.ShapeDtypeStruct((N, D), jnp.float32),
        scratch_shapes=[
            pltpu.VMEM((QBLK, H * hd), jnp.bfloat16),
        ],
    )(q, k, vx, node_c, node_r, Wo.astype(jnp.bfloat16))
    return out.reshape(B, N, D)


# speedup vs baseline: 1.0430x; 1.0430x over previous
import functools
import math

import jax
import jax.numpy as jnp
from jax import lax
from jax.experimental import pallas as pl
from jax.experimental.pallas import tpu as pltpu
from jax.experimental.pallas import tpu_sc as plsc

EMBED_DIM = 768
NUM_HEADS = 12
HEAD_DIM = 64
NUM_NODES = 8
ROPE_BASE = 10000.0

QBLK = 256

_TWO_PI_HI = 6.28125
_TWO_PI_LO = 0.0019353071795864769
_INV_TWO_PI = 1.0 / (2.0 * math.pi)

_COS_COEF = [1.0 / math.factorial(2 * m) * (-1) ** m for m in range(9)]
_SIN_COEF = [1.0 / math.factorial(2 * m + 1) * (-1) ** m for m in range(9)]


def _cos_sin(x):
    f32 = jnp.float32
    half = x.shape[1]
    x2 = jnp.concatenate([x, x], axis=1)
    u = x2 * _INV_TWO_PI
    kq = jnp.floor(u + 0.5)
    r = (x2 - kq * _TWO_PI_HI) - kq * _TWO_PI_LO
    y = r * r
    lane = jax.lax.broadcasted_iota(jnp.int32, (1, 2 * half), 1)
    is_cos = lane < half
    coef = [jnp.where(is_cos, _COS_COEF[m], _SIN_COEF[m]) for m in range(9)]
    p = jnp.broadcast_to(coef[8], y.shape)
    for m in range(7, -1, -1):
        p = p * y + coef[m]
    p = p * jnp.where(is_cos, 1.0, r)
    return p[:, :half].astype(f32), p[:, half:].astype(f32)


def _table_kernel(tab_ref):
    f32 = jnp.float32
    n = tab_ref.shape[0]
    hd = HEAD_DIM
    half = hd // 2
    t = jax.lax.broadcasted_iota(jnp.int32, (n, 1), 0).astype(f32)
    i2 = jax.lax.broadcasted_iota(jnp.int32, (1, half), 1).astype(f32)
    inv_freq = jnp.exp(i2 * (-2.0 * math.log(ROPE_BASE) / hd))
    cos, sin = _cos_sin(t * inv_freq)
    tab_ref[...] = jnp.concatenate([cos, cos, sin, sin], axis=1)


def _route_kernel(x_ref, wr_ref, nc_out, nr_out, pos_out):
    f32 = jnp.float32
    x = x_ref[...]
    n = x.shape[0]
    K = NUM_NODES
    H = NUM_HEADS

    logits_t = jax.lax.dot_general(wr_ref[...], x, (((1,), (1,)), ((), ())),
                                   preferred_element_type=f32)
    kidx = jax.lax.broadcasted_iota(jnp.int32, (K, n), 0).astype(f32)
    onehots = []
    nodes = []
    for h in range(H):
        lt = logits_t[h * K:(h + 1) * K]
        mx = jnp.max(lt, axis=0, keepdims=True)
        node_t = jnp.min(jnp.where(lt == mx, kidx, float(K)),
                         axis=0, keepdims=True)
        nodes.append(node_t)
        onehots.append((kidx == node_t).astype(f32))
    oh = jnp.concatenate(onehots, axis=0)
    cum = oh
    shift = 1
    while shift < n:
        zeros = jnp.zeros((H * K, shift), dtype=f32)
        cum = cum + jnp.concatenate([zeros, cum[:, :-shift]], axis=1)
        shift *= 2
    sel = oh * cum
    for h in range(H):
        pos_t = jnp.sum(sel[h * K:(h + 1) * K], axis=0, keepdims=True) - 1.0
        nr_out[h] = nodes[h]
        nc_out[h] = jnp.transpose(nodes[h])
        pos_out[h] = pos_t.astype(jnp.int32)


def _qkv_kernel(x_ref, wqkv_ref, qk_out, vx_out):
    f32 = jnp.float32
    bf16 = jnp.bfloat16
    x = x_ref[...]
    n = x.shape[0]
    hd = HEAD_DIM
    qkv = jax.lax.dot_general(x.astype(bf16), wqkv_ref[0].astype(bf16),
                              (((1,), (1,)), ((), ())),
                              preferred_element_type=f32)
    qk_out[0] = qkv[:, :2 * hd]
    vx_out[0] = jnp.concatenate([qkv[:, 2 * hd:], jnp.ones((n, hd), f32)],
                                axis=1).astype(bf16)


def _make_sc_gather(n_rows, width, n_workers):
    rows_per_w = n_rows // n_workers
    mesh = plsc.VectorSubcoreMesh(core_axis_name="c", subcore_axis_name="s")

    @functools.partial(
        pl.kernel, mesh=mesh,
        out_type=jax.ShapeDtypeStruct((n_rows, width), jnp.float32),
        scratch_types=[
            pltpu.VMEM((rows_per_w,), jnp.int32),
            pltpu.VMEM((rows_per_w, width), jnp.float32),
            pltpu.SemaphoreType.DMA,
        ],
    )
    def sc_gather(table_hbm, idx_hbm, out_hbm, idx_v, rows_v, sem):
        wid = lax.axis_index("s") * 2 + lax.axis_index("c")
        base = wid * rows_per_w
        pltpu.sync_copy(idx_hbm.at[pl.ds(base, rows_per_w)], idx_v)
        pltpu.async_copy(table_hbm.at[idx_v], rows_v, sem).wait()
        pltpu.sync_copy(rows_v, out_hbm.at[pl.ds(base, rows_per_w)])

    return sc_gather


def _rope_apply_kernel(qk_ref, cs_ref, q_out, k_out):
    bf16 = jnp.bfloat16
    hd = HEAD_DIM
    half = hd // 2
    scale = hd ** -0.5
    qk = qk_ref[0]
    cs = cs_ref[0]
    cos2 = cs[:, :hd]
    sin2 = cs[:, hd:]
    cos4 = jnp.concatenate([cos2, cos2], axis=1)
    sin4 = jnp.concatenate([sin2, sin2], axis=1)
    rot = jnp.concatenate([-qk[:, half:hd], qk[:, :half],
                           -qk[:, hd + half:], qk[:, hd:hd + half]], axis=1)
    qk_roped = qk * cos4 + rot * sin4
    q_out[0] = (qk_roped[:, :hd] * scale).astype(bf16)
    k_out[0] = qk_roped[:, hd:].astype(bf16)


def _attn_out_kernel(q_ref, k_ref, vx_ref, nc_ref, nr_ref, wo_ref,
                     o_ref, y_ref):
    f32 = jnp.float32
    bf16 = jnp.bfloat16
    qi = pl.program_id(0)
    H = q_ref.shape[0]
    n = k_ref.shape[1]
    hd = HEAD_DIM

    rr = jax.lax.broadcasted_iota(jnp.int32, (QBLK, n), 0) + qi * QBLK
    cc = jax.lax.broadcasted_iota(jnp.int32, (QBLK, n), 1)
    causal = rr >= cc

    for h in range(H):
        s = jax.lax.dot_general(q_ref[h], k_ref[h], (((1,), (1,)), ((), ())),
                                preferred_element_type=f32)
        d = nc_ref[h] - nr_ref[h]
        e = jnp.exp(s + d * d * (-1e9))
        e = jnp.where(causal, e, 0.0)
        accx = jax.lax.dot_general(e.astype(bf16), vx_ref[h],
                                   (((1,), (0,)), ((), ())),
                                   preferred_element_type=f32)
        y_ref[:, h * hd:(h + 1) * hd] = (
            accx[:, :hd] / accx[:, hd:hd + 1]).astype(bf16)

    o_ref[...] = jax.lax.dot_general(y_ref[...], wo_ref[...],
                                     (((1,), (1,)), ((), ())),
                                     preferred_element_type=f32)


@jax.jit
def kernel(x, Wq, Wk, Wv, Wr, Wo):
    B, N, D = x.shape
    H, hd, K = NUM_HEADS, HEAD_DIM, NUM_NODES
    x2 = x.reshape(N, D)
    wqkv = jnp.concatenate([Wq.reshape(H, hd, D), Wk.reshape(H, hd, D),
                            Wv.reshape(H, hd, D)], axis=1)

    table = pl.pallas_call(
        _table_kernel,
        out_shape=jax.ShapeDtypeStruct((N, 2 * hd), jnp.float32),
    )()

    node_c, node_r, pos = pl.pallas_call(
        _route_kernel,
        out_specs=[
            pl.BlockSpec((H, N, 1), lambda: (0, 0, 0)),
            pl.BlockSpec((H, 1, N), lambda: (0, 0, 0)),
            pl.BlockSpec((H, 1, N), lambda: (0, 0, 0)),
        ],
        out_shape=[
            jax.ShapeDtypeStruct((H, N, 1), jnp.float32),
            jax.ShapeDtypeStruct((H, 1, N), jnp.float32),
            jax.ShapeDtypeStruct((H, 1, N), jnp.int32),
        ],
    )(x2, Wr)

    sc_gather = _make_sc_gather(H * N, 2 * hd, 32)
    cossin = sc_gather(table, pos.reshape(H * N))

    qk2, vx = pl.pallas_call(
        _qkv_kernel,
        grid=(H,),
        in_specs=[
            pl.BlockSpec((N, D), lambda h: (0, 0)),
            pl.BlockSpec((1, 3 * hd, D), lambda h: (h, 0, 0)),
        ],
        out_specs=[
            pl.BlockSpec((1, N, 2 * hd), lambda h: (h, 0, 0)),
            pl.BlockSpec((1, N, 2 * hd), lambda h: (h, 0, 0)),
        ],
        out_shape=[
            jax.ShapeDtypeStruct((H, N, 2 * hd), jnp.float32),
            jax.ShapeDtypeStruct((H, N, 2 * hd), jnp.bfloat16),
        ],
    )(x2, wqkv)

    q, k = pl.pallas_call(
        _rope_apply_kernel,
        grid=(H,),
        in_specs=[
            pl.BlockSpec((1, N, 2 * hd), lambda h: (h, 0, 0)),
            pl.BlockSpec((1, N, 2 * hd), lambda h: (h, 0, 0)),
        ],
        out_specs=[
            pl.BlockSpec((1, N, hd), lambda h: (h, 0, 0)),
            pl.BlockSpec((1, N, hd), lambda h: (h, 0, 0)),
        ],
        out_shape=[
            jax.ShapeDtypeStruct((H, N, hd), jnp.bfloat16),
            jax.ShapeDtypeStruct((H, N, hd), jnp.bfloat16),
        ],
    )(qk2, cossin.reshape(H, N, 2 * hd))

    out = pl.pallas_call(
        _attn_out_kernel,
        grid=(N // QBLK,),
        in_specs=[
            pl.BlockSpec((H, QBLK, hd), lambda i: (0, i, 0)),
            pl.BlockSpec((H, N, hd), lambda i: (0, 0, 0)),
            pl.BlockSpec((H, N, 2 * hd), lambda i: (0, 0, 0)),
            pl.BlockSpec((H, QBLK, 1), lambda i: (0, i, 0)),
            pl.BlockSpec((H, 1, N), lambda i: (0, 0, 0)),
            pl.BlockSpec((D, H * hd), lambda i: (0, 0)),
        ],
        out_specs=pl.BlockSpec((QBLK, D), lambda i: (i, 0)),
        out_shape=jax.ShapeDtypeStruct((N, D), jnp.float32),
        scratch_shapes=[
            pltpu.VMEM((QBLK, H * hd), jnp.bfloat16),
        ],
    )(q, k, vx, node_c, node_r, Wo.astype(jnp.bfloat16))
    return out.reshape(B, N, D)


# speedup vs baseline: 1.2220x; 1.1717x over previous
import functools
import math

import jax
import jax.numpy as jnp
from jax.experimental import pallas as pl
from jax.experimental.pallas import tpu as pltpu

EMBED_DIM = 768
NUM_HEADS = 12
HEAD_DIM = EMBED_DIM // NUM_HEADS
NUM_NODES = 8
ROPE_BASE = 10000.0

QBLK = 256
KBLK = 256

_TWO_PI_HI = 6.28125
_TWO_PI_LO = 0.0019353071795864769
_INV_TWO_PI = 1.0 / (2.0 * math.pi)

_COS_COEF = [1.0 / math.factorial(2 * m) * (-1) ** m for m in range(9)]
_SIN_COEF = [1.0 / math.factorial(2 * m + 1) * (-1) ** m for m in range(9)]


def _cos_sin(x):
    f32 = jnp.float32
    half = x.shape[1]
    x2 = jnp.concatenate([x, x], axis=1)
    u = x2 * _INV_TWO_PI
    kq = jnp.floor(u + 0.5)
    r = (x2 - kq * _TWO_PI_HI) - kq * _TWO_PI_LO
    y = r * r
    lane = jax.lax.broadcasted_iota(jnp.int32, (1, 2 * half), 1)
    is_cos = lane < half
    coef = [jnp.where(is_cos, _COS_COEF[m], _SIN_COEF[m]) for m in range(9)]
    p = jnp.broadcast_to(coef[8], y.shape)
    for m in range(7, -1, -1):
        p = p * y + coef[m]
    p = p * jnp.where(is_cos, 1.0, r)
    return p[:, :half].astype(f32), p[:, half:].astype(f32)


def _proj_route_kernel(x_ref, wqkv_ref, wr_ref,
                       q_out, k_out, vx_out, nc_out, nr_out):
    f32 = jnp.float32
    bf16 = jnp.bfloat16
    x = x_ref[...]
    xb = x.astype(bf16)
    n = x.shape[0]
    K = wr_ref.shape[1]
    hd = HEAD_DIM
    scale = hd ** -0.5

    qkv = jax.lax.dot_general(xb, wqkv_ref[0].astype(bf16),
                              (((1,), (1,)), ((), ())),
                              preferred_element_type=f32)
    logits_t = jax.lax.dot_general(wr_ref[0], x, (((1,), (1,)), ((), ())),
                                   preferred_element_type=f32)

    kidx = jax.lax.broadcasted_iota(jnp.int32, (K, n), 0).astype(f32)
    mx = jnp.max(logits_t, axis=0, keepdims=True)
    node_t = jnp.min(jnp.where(logits_t == mx, kidx, float(K)),
                     axis=0, keepdims=True)
    onehot_t = (kidx == node_t).astype(f32)
    cum = onehot_t
    shift = 1
    while shift < n:
        zeros = jnp.zeros((K, shift), dtype=f32)
        cum = cum + jnp.concatenate([zeros, cum[:, :-shift]], axis=1)
        shift *= 2
    pos_t = jnp.sum(onehot_t * cum, axis=0, keepdims=True) - 1.0
    pos = jnp.transpose(pos_t)

    half = hd // 2
    i2 = jax.lax.broadcasted_iota(jnp.int32, (1, half), 1).astype(f32)
    inv_freq = jnp.exp(i2 * (-2.0 * math.log(ROPE_BASE) / hd))
    ang = pos * inv_freq
    cos, sin = _cos_sin(ang)
    cos4 = jnp.concatenate([cos, cos, cos, cos], axis=1)
    sin4 = jnp.concatenate([sin, sin, sin, sin], axis=1)

    qk = qkv[:, :2 * hd]
    rot = jnp.concatenate([-qk[:, half:hd], qk[:, :half],
                           -qk[:, hd + half:], qk[:, hd:hd + half]], axis=1)
    qk_roped = qk * cos4 + rot * sin4
    q_out[0] = (qk_roped[:, :hd] * scale).astype(bf16)
    k_out[0] = qk_roped[:, hd:].astype(bf16)
    vx_out[0] = jnp.concatenate([qkv[:, 2 * hd:], jnp.ones((n, hd), f32)],
                                axis=1).astype(bf16)
    nr_out[0] = node_t
    nc_out[0] = jnp.transpose(node_t)


def _attn_out_kernel(q_ref, k_ref, vx_ref, nc_ref, nr_ref, wo_ref,
                     o_ref, y_ref):
    f32 = jnp.float32
    bf16 = jnp.bfloat16
    qi = pl.program_id(0)
    H = q_ref.shape[0]
    n = k_ref.shape[1]
    hd = HEAD_DIM

    rr = jax.lax.broadcasted_iota(jnp.int32, (QBLK, n), 0) + qi * QBLK
    cc = jax.lax.broadcasted_iota(jnp.int32, (QBLK, n), 1)
    cb = jnp.where(rr >= cc, 0.0, 1.0)

    for h in range(H):
        s = jax.lax.dot_general(q_ref[h], k_ref[h], (((1,), (1,)), ((), ())),
                                preferred_element_type=f32)
        d = nc_ref[h] - nr_ref[h]
        t = d * d + cb
        e = jnp.exp(t * (-1e9) + s)
        accx = jax.lax.dot_general(e.astype(bf16), vx_ref[h],
                                   (((1,), (0,)), ((), ())),
                                   preferred_element_type=f32)
        y_ref[:, h * hd:(h + 1) * hd] = (
            accx[:, :hd] / accx[:, hd:hd + 1]).astype(bf16)

    o_ref[...] = jax.lax.dot_general(y_ref[...], wo_ref[...],
                                     (((1,), (1,)), ((), ())),
                                     preferred_element_type=f32)


@jax.jit
def kernel(x, Wq, Wk, Wv, Wr, Wo):
    B, N, D = x.shape
    H, hd, K = NUM_HEADS, HEAD_DIM, NUM_NODES
    x2 = x.reshape(N, D)
    wqkv = jnp.concatenate([Wq.reshape(H, hd, D), Wk.reshape(H, hd, D),
                            Wv.reshape(H, hd, D)], axis=1)

    q, k, vx, node_c, node_r = pl.pallas_call(
        _proj_route_kernel,
        grid=(H,),
        in_specs=[
            pl.BlockSpec((N, D), lambda h: (0, 0)),
            pl.BlockSpec((1, 3 * hd, D), lambda h: (h, 0, 0)),
            pl.BlockSpec((1, K, D), lambda h: (h, 0, 0)),
        ],
        out_specs=[
            pl.BlockSpec((1, N, hd), lambda h: (h, 0, 0)),
            pl.BlockSpec((1, N, hd), lambda h: (h, 0, 0)),
            pl.BlockSpec((1, N, 2 * hd), lambda h: (h, 0, 0)),
            pl.BlockSpec((1, N, 1), lambda h: (h, 0, 0)),
            pl.BlockSpec((1, 1, N), lambda h: (h, 0, 0)),
        ],
        out_shape=[
            jax.ShapeDtypeStruct((H, N, hd), jnp.bfloat16),
            jax.ShapeDtypeStruct((H, N, hd), jnp.bfloat16),
            jax.ShapeDtypeStruct((H, N, 2 * hd), jnp.bfloat16),
            jax.ShapeDtypeStruct((H, N, 1), jnp.float32),
            jax.ShapeDtypeStruct((H, 1, N), jnp.float32),
        ],
    )(x2, wqkv, Wr.reshape(H, K, D))

    out = pl.pallas_call(
        _attn_out_kernel,
        grid=(N // QBLK,),
        in_specs=[
            pl.BlockSpec((H, QBLK, hd), lambda i: (0, i, 0)),
            pl.BlockSpec((H, N, hd), lambda i: (0, 0, 0)),
            pl.BlockSpec((H, N, 2 * hd), lambda i: (0, 0, 0)),
            pl.BlockSpec((H, QBLK, 1), lambda i: (0, i, 0)),
            pl.BlockSpec((H, 1, N), lambda i: (0, 0, 0)),
            pl.BlockSpec((D, H * hd), lambda i: (0, 0)),
        ],
        out_specs=pl.BlockSpec((QBLK, D), lambda i: (i, 0)),
        out_shape=jax.ShapeDtypeStruct((N, D), jnp.float32),
        scratch_shapes=[
            pltpu.VMEM((QBLK, H * hd), jnp.bfloat16),
        ],
    )(q, k, vx, node_c, node_r, Wo.astype(jnp.bfloat16))
    return out.reshape(B, N, D)
